# Initial kernel scaffold; baseline (speedup 1.0000x reference)
import functools

import jax
import jax.numpy as jnp
from jax import lax
from jax.experimental import pallas as pl
from jax.experimental.pallas import tpu as pltpu
from jax.experimental.pallas import tpu_sc as plsc

N = 10000
E = 320000
D = 128

NC = 2
NS = 16
NW = NC * NS
CHUNK = 128
EPT_CHUNKS = -(-E // (NW * CHUNK))
EPT = EPT_CHUNKS * CHUNK
EPAD = EPT * NW
NPAD = 10016
RPT = NPAD // NS
DEGW = 16

_MESH = plsc.VectorSubcoreMesh(core_axis_name="c", subcore_axis_name="s")


def _sc_agg_body(with_deg, *refs):
    if with_deg:
        (y, srcp, dstp, zrow, zdeg, onesd, out_acc, out_deg,
         acc_sh, deg_sh, sidx, didx, rows, ones_v, sem) = refs
    else:
        (y, srcp, dstp, zrow, out_acc,
         acc_sh, sidx, didx, rows, sem) = refs

    c = lax.axis_index("c")
    s = lax.axis_index("s")
    wid = c * NS + s

    r0 = s * RPT
    pltpu.sync_copy(zrow.at[pl.ds(r0, RPT)], acc_sh.at[pl.ds(r0, RPT)])
    if with_deg:
        pltpu.sync_copy(zdeg.at[pl.ds(r0, RPT)], deg_sh.at[pl.ds(r0, RPT)])
        pltpu.sync_copy(onesd, ones_v)
    plsc.subcore_barrier()

    base = wid * EPT

    def chunk_body(i, carry):
        off = base + i * CHUNK
        pltpu.sync_copy(srcp.at[pl.ds(off, CHUNK)], sidx)
        pltpu.sync_copy(dstp.at[pl.ds(off, CHUNK)], didx)
        pltpu.async_copy(y.at[sidx], rows, sem).wait()
        pltpu.sync_copy(rows, acc_sh.at[didx], add=True)
        if with_deg:
            pltpu.sync_copy(ones_v, deg_sh.at[didx], add=True)
        return carry

    lax.fori_loop(0, EPT_CHUNKS, chunk_body, 0)
    plsc.subcore_barrier()

    pltpu.sync_copy(acc_sh.at[pl.ds(r0, RPT)], out_acc.at[c, pl.ds(r0, RPT)])
    if with_deg:
        pltpu.sync_copy(deg_sh.at[pl.ds(r0, RPT)], out_deg.at[c, pl.ds(r0, RPT)])


_sc_agg_deg = functools.partial(
    pl.kernel,
    functools.partial(_sc_agg_body, True),
    out_type=[
        jax.ShapeDtypeStruct((NC, NPAD, D), jnp.float32),
        jax.ShapeDtypeStruct((NC, NPAD, DEGW), jnp.float32),
    ],
    mesh=_MESH,
    scratch_types=[
        pltpu.VMEM_SHARED((NPAD, D), jnp.float32),
        pltpu.VMEM_SHARED((NPAD, DEGW), jnp.float32),
        pltpu.VMEM((CHUNK,), jnp.int32),
        pltpu.VMEM((CHUNK,), jnp.int32),
        pltpu.VMEM((CHUNK, D), jnp.float32),
        pltpu.VMEM((CHUNK, DEGW), jnp.float32),
        pltpu.SemaphoreType.DMA,
    ],
)()

_sc_agg = functools.partial(
    pl.kernel,
    functools.partial(_sc_agg_body, False),
    out_type=jax.ShapeDtypeStruct((NC, NPAD, D), jnp.float32),
    mesh=_MESH,
    scratch_types=[
        pltpu.VMEM_SHARED((NPAD, D), jnp.float32),
        pltpu.VMEM((CHUNK,), jnp.int32),
        pltpu.VMEM((CHUNK,), jnp.int32),
        pltpu.VMEM((CHUNK, D), jnp.float32),
        pltpu.SemaphoreType.DMA,
    ],
)()


BM = 2000


def _tc_in_body(x_ref, ws_ref, wn_ref, b_ref, z_ref, y_ref):
    x = x_ref[...]
    z_ref[...] = (
        jnp.dot(x, ws_ref[...], preferred_element_type=jnp.float32) + b_ref[...]
    )
    y_ref[...] = jnp.dot(x, wn_ref[...], preferred_element_type=jnp.float32)


def _tc_mid_body(z1_ref, acc_ref, deg_ref, ws_ref, wn_ref, b_ref, z2_ref, y2_ref):
    agg = acc_ref[0] + acc_ref[1]
    deg = jnp.maximum(deg_ref[0][:, 0:1] + deg_ref[1][:, 0:1], 1.0)
    h = jnp.maximum(z1_ref[...] + agg / deg, 0.0)
    z2_ref[...] = (
        jnp.dot(h, ws_ref[...], preferred_element_type=jnp.float32) + b_ref[...]
    )
    y2_ref[...] = jnp.dot(h, wn_ref[...], preferred_element_type=jnp.float32)


def _tc_out_body(z2_ref, acc_ref, deg_ref, out_ref):
    agg = acc_ref[0] + acc_ref[1]
    deg = jnp.maximum(deg_ref[0][:, 0:1] + deg_ref[1][:, 0:1], 1.0)
    out_ref[...] = z2_ref[...] + agg / deg


_row_spec = pl.BlockSpec((BM, D), lambda i: (i, 0))
_acc_spec = pl.BlockSpec((NC, BM, D), lambda i: (0, i, 0))
_deg_spec = pl.BlockSpec((NC, BM, DEGW), lambda i: (0, i, 0))
_w_spec = pl.BlockSpec((D, D), lambda i: (0, 0))
_b_spec = pl.BlockSpec((1, D), lambda i: (0, 0))

_tc_in = pl.pallas_call(
    _tc_in_body,
    grid=(N // BM,),
    in_specs=[_row_spec, _w_spec, _w_spec, _b_spec],
    out_specs=[_row_spec, _row_spec],
    out_shape=[
        jax.ShapeDtypeStruct((N, D), jnp.float32),
        jax.ShapeDtypeStruct((N, D), jnp.float32),
    ],
)

_tc_mid = pl.pallas_call(
    _tc_mid_body,
    grid=(N // BM,),
    in_specs=[_row_spec, _acc_spec, _deg_spec, _w_spec, _w_spec, _b_spec],
    out_specs=[_row_spec, _row_spec],
    out_shape=[
        jax.ShapeDtypeStruct((N, D), jnp.float32),
        jax.ShapeDtypeStruct((N, D), jnp.float32),
    ],
)

_tc_out = pl.pallas_call(
    _tc_out_body,
    grid=(N // BM,),
    in_specs=[_row_spec, _acc_spec, _deg_spec],
    out_specs=_row_spec,
    out_shape=jax.ShapeDtypeStruct((N, D), jnp.float32),
)


@jax.jit
def kernel(x, edge_index, W1_self, W1_neigh, b1, W2_self, W2_neigh, b2):
    src = edge_index[0]
    dst = edge_index[1]
    pad = EPAD - E
    srcp = jnp.pad(src, (0, pad))
    dstp = jnp.pad(dst, (0, pad), constant_values=N)
    zrow = jnp.zeros((NPAD, D), jnp.float32)
    zdeg = jnp.zeros((NPAD, DEGW), jnp.float32)
    onesd = jnp.ones((CHUNK, DEGW), jnp.float32)

    z1, y1 = _tc_in(x, W1_self, W1_neigh, b1.reshape(1, D))
    acc1, deg = _sc_agg_deg(y1, srcp, dstp, zrow, zdeg, onesd)
    acc1 = acc1[:, :N]
    degn = deg[:, :N]
    z2, y2 = _tc_mid(z1, acc1, degn, W2_self, W2_neigh, b2.reshape(1, D))
    acc2 = _sc_agg(y2, srcp, dstp, zrow)
    return _tc_out(z2, acc2[:, :N], degn)


# speedup vs baseline: 4.3230x; 4.3230x over previous
import functools

import jax
import jax.numpy as jnp
from jax import lax
from jax.experimental import pallas as pl
from jax.experimental.pallas import tpu as pltpu
from jax.experimental.pallas import tpu_sc as plsc

N = 10000
E = 320000
D = 128

NC = 2
NS = 16
NW = NC * NS
CHUNK = 128
EPT_CHUNKS = -(-E // (NW * CHUNK))
EPT = EPT_CHUNKS * CHUNK
EPAD = EPT * NW
NPAD = 10112
RPT = NPAD // NS


_MESH = plsc.VectorSubcoreMesh(core_axis_name="c", subcore_axis_name="s")


_ZOFFS = [0, 128, 256, 384, RPT - CHUNK]


def _sc_agg_body(with_deg, *refs):
    if with_deg:
        (y, srcp, dstp, zrow, zdeg, onesd, out_acc, out_deg,
         acc_sh, deg_sh, sidx, didx, rows, ones_v, sem) = refs
    else:
        (y, srcp, dstp, zrow, out_acc,
         acc_sh, sidx, didx, rows, sem) = refs

    c = lax.axis_index("c")
    s = lax.axis_index("s")
    wid = c * NS + s

    r0 = s * RPT
    pltpu.sync_copy(zrow, rows)
    for o in _ZOFFS:
        pltpu.sync_copy(rows, acc_sh.at[pl.ds(r0 + o, CHUNK)])
    if with_deg:
        pltpu.sync_copy(zdeg, ones_v)
        for o in _ZOFFS:
            pltpu.sync_copy(ones_v, deg_sh.at[pl.ds(r0 + o, CHUNK)])
        pltpu.sync_copy(onesd, ones_v)
    plsc.subcore_barrier()

    base = wid * EPT

    def chunk_body(i, carry):
        off = base + i * CHUNK
        pltpu.sync_copy(srcp.at[pl.ds(off, CHUNK)], sidx)
        pltpu.sync_copy(dstp.at[pl.ds(off, CHUNK)], didx)
        pltpu.async_copy(y.at[sidx], rows, sem).wait()
        pltpu.sync_copy(rows, acc_sh.at[didx], add=True)
        if with_deg:
            pltpu.sync_copy(ones_v, deg_sh.at[didx], add=True)
        return carry

    lax.fori_loop(0, EPT_CHUNKS, chunk_body, 0)
    plsc.subcore_barrier()

    ob = c * NPAD + r0
    for o in _ZOFFS:
        pltpu.sync_copy(acc_sh.at[pl.ds(r0 + o, CHUNK)], rows)
        pltpu.sync_copy(rows, out_acc.at[pl.ds(ob + o, CHUNK)])
    if with_deg:
        for o in _ZOFFS:
            pltpu.sync_copy(deg_sh.at[pl.ds(r0 + o, CHUNK)], ones_v)
            pltpu.sync_copy(ones_v, out_deg.at[pl.ds(ob + o, CHUNK)])


_sc_agg_deg = functools.partial(
    pl.kernel,
    functools.partial(_sc_agg_body, True),
    out_type=[
        jax.ShapeDtypeStruct((NC * NPAD, D), jnp.float32),
        jax.ShapeDtypeStruct((NC * NPAD,), jnp.float32),
    ],
    mesh=_MESH,
    scratch_types=[
        pltpu.VMEM_SHARED((NPAD, D), jnp.float32),
        pltpu.VMEM_SHARED((NPAD,), jnp.float32),
        pltpu.VMEM((CHUNK,), jnp.int32),
        pltpu.VMEM((CHUNK,), jnp.int32),
        pltpu.VMEM((CHUNK, D), jnp.float32),
        pltpu.VMEM((CHUNK,), jnp.float32),
        pltpu.SemaphoreType.DMA,
    ],
)()

_sc_agg = functools.partial(
    pl.kernel,
    functools.partial(_sc_agg_body, False),
    out_type=jax.ShapeDtypeStruct((NC * NPAD, D), jnp.float32),
    mesh=_MESH,
    scratch_types=[
        pltpu.VMEM_SHARED((NPAD, D), jnp.float32),
        pltpu.VMEM((CHUNK,), jnp.int32),
        pltpu.VMEM((CHUNK,), jnp.int32),
        pltpu.VMEM((CHUNK, D), jnp.float32),
        pltpu.SemaphoreType.DMA,
    ],
)()


BM = 2000


def _tc_in_body(x_ref, ws_ref, wn_ref, b_ref, z_ref, y_ref):
    x = x_ref[...]
    z_ref[...] = (
        jnp.dot(x, ws_ref[...], preferred_element_type=jnp.float32) + b_ref[...]
    )
    y_ref[...] = jnp.dot(x, wn_ref[...], preferred_element_type=jnp.float32)


def _tc_mid_body(z1_ref, acc_ref, dg0_ref, dg1_ref, ws_ref, wn_ref, b_ref, z2_ref, y2_ref):
    agg = acc_ref[0] + acc_ref[1]
    deg = jnp.maximum(dg0_ref[...] + dg1_ref[...], 1.0)
    h = jnp.maximum(z1_ref[...] + agg / deg, 0.0)
    z2_ref[...] = (
        jnp.dot(h, ws_ref[...], preferred_element_type=jnp.float32) + b_ref[...]
    )
    y2_ref[...] = jnp.dot(h, wn_ref[...], preferred_element_type=jnp.float32)


def _tc_out_body(z2_ref, acc_ref, dg0_ref, dg1_ref, out_ref):
    agg = acc_ref[0] + acc_ref[1]
    deg = jnp.maximum(dg0_ref[...] + dg1_ref[...], 1.0)
    out_ref[...] = z2_ref[...] + agg / deg


_row_spec = pl.BlockSpec((BM, D), lambda i: (i, 0))
_acc_spec = pl.BlockSpec((NC, BM, D), lambda i: (0, i, 0))
_deg_spec = pl.BlockSpec((BM, 1), lambda i: (i, 0))
_w_spec = pl.BlockSpec((D, D), lambda i: (0, 0))
_b_spec = pl.BlockSpec((1, D), lambda i: (0, 0))

_tc_in = pl.pallas_call(
    _tc_in_body,
    grid=(N // BM,),
    in_specs=[_row_spec, _w_spec, _w_spec, _b_spec],
    out_specs=[_row_spec, _row_spec],
    out_shape=[
        jax.ShapeDtypeStruct((N, D), jnp.float32),
        jax.ShapeDtypeStruct((N, D), jnp.float32),
    ],
)

_tc_mid = pl.pallas_call(
    _tc_mid_body,
    grid=(N // BM,),
    in_specs=[_row_spec, _acc_spec, _deg_spec, _deg_spec, _w_spec, _w_spec, _b_spec],
    out_specs=[_row_spec, _row_spec],
    out_shape=[
        jax.ShapeDtypeStruct((N, D), jnp.float32),
        jax.ShapeDtypeStruct((N, D), jnp.float32),
    ],
)

_tc_out = pl.pallas_call(
    _tc_out_body,
    grid=(N // BM,),
    in_specs=[_row_spec, _acc_spec, _deg_spec, _deg_spec],
    out_specs=_row_spec,
    out_shape=jax.ShapeDtypeStruct((N, D), jnp.float32),
)


@jax.jit
def kernel(x, edge_index, W1_self, W1_neigh, b1, W2_self, W2_neigh, b2):
    src = edge_index[0]
    dst = edge_index[1]
    pad = EPAD - E
    srcp = jnp.pad(src, (0, pad))
    dstp = jnp.pad(dst, (0, pad), constant_values=N)
    zrow = jnp.zeros((CHUNK, D), jnp.float32)
    zdeg = jnp.zeros((CHUNK,), jnp.float32)
    onesd = jnp.ones((CHUNK,), jnp.float32)

    z1, y1 = _tc_in(x, W1_self, W1_neigh, b1.reshape(1, D))
    acc1, deg = _sc_agg_deg(y1, srcp, dstp, zrow, zdeg, onesd)
    acc1 = acc1.reshape(NC, NPAD, D)[:, :N]
    degn = deg.reshape(NC, NPAD)[:, :N]
    dg0 = degn[0][:, None]
    dg1 = degn[1][:, None]
    z2, y2 = _tc_mid(z1, acc1, dg0, dg1, W2_self, W2_neigh, b2.reshape(1, D))
    acc2 = _sc_agg(y2, srcp, dstp, zrow)
    return _tc_out(z2, acc2.reshape(NC, NPAD, D)[:, :N], dg0, dg1)
